# Initial kernel scaffold; baseline (speedup 1.0000x reference)
import jax
import jax.numpy as jnp
from jax.experimental import pallas as pl


def kernel(x, edge_index, batch, W_l, b_l, W_r, W1, b1, W2, b2):
    raise NotImplementedError("write your pallas kernel here")


# speedup vs baseline: 5.4523x; 5.4523x over previous
import functools

import jax
import jax.numpy as jnp
from jax import lax
from jax.experimental import pallas as pl
from jax.experimental.pallas import tpu as pltpu
from jax.experimental.pallas import tpu_sc as plsc

NUM_G = 64
CHUNK = 128
NPAD = 10240
NW = 32


def _sc_agg(x, src2, dst2):
    n, d = x.shape
    nch = src2.shape[0]
    rpt = NPAD // 16

    mesh = plsc.VectorSubcoreMesh(core_axis_name="c", subcore_axis_name="s")

    @functools.partial(
        pl.kernel,
        mesh=mesh,
        out_type=(
            jax.ShapeDtypeStruct((2 * NPAD, d), jnp.float32),
            jax.ShapeDtypeStruct((NW, NPAD // 128, 128), jnp.float32),
        ),
        scratch_types=[
            pltpu.VMEM_SHARED((NPAD, d), jnp.float32),
            pltpu.VMEM((NPAD // 128, 128), jnp.float32),
            pltpu.VMEM((CHUNK,), jnp.int32),
            pltpu.VMEM((CHUNK,), jnp.int32),
            pltpu.VMEM((CHUNK, d), jnp.float32),
            pltpu.SemaphoreType.DMA,
        ],
        compiler_params=pltpu.CompilerParams(
            use_tc_tiling_on_sc=False, needs_layout_passes=False),
    )
    def k(x_hbm, src_hbm, dst_hbm, part_hbm, cnt_hbm,
          acc_sh, hist, sidx, didx, rows, sem):
        c = lax.axis_index("c")
        s = lax.axis_index("s")
        wid = s * 2 + c

        def fillz(i, carry):
            for j in range(d // 16):
                rows[i, pl.ds(j * 16, 16)] = jnp.zeros((16,), jnp.float32)
            return carry

        lax.fori_loop(0, CHUNK, fillz, 0)

        def fillh(i, carry):
            for j in range(128 // 16):
                hist[i, pl.ds(j * 16, 16)] = jnp.zeros((16,), jnp.float32)
            return carry

        lax.fori_loop(0, NPAD // 128, fillh, 0)

        r0 = s * rpt
        for j in range(rpt // CHUNK):
            pltpu.sync_copy(rows, acc_sh.at[pl.ds(r0 + j * CHUNK, CHUNK)])
        plsc.subcore_barrier()

        nk = (nch + 31) // 32
        ones16 = jnp.ones((16,), jnp.float32)

        def body(kk, carry):
            cid = wid + 32 * kk

            @pl.when(cid < nch)
            def _():
                pltpu.sync_copy(src_hbm.at[cid], sidx)
                pltpu.sync_copy(dst_hbm.at[cid], didx)
                for j in range(CHUNK // 16):
                    dj = didx[pl.ds(j * 16, 16)]
                    plsc.addupdate_scatter(
                        hist, [dj >> 7, dj & 127], ones16)
                pltpu.async_copy(x_hbm.at[sidx], rows, sem).wait()
                pltpu.sync_copy(rows, acc_sh.at[didx], add=True)

            return carry

        lax.fori_loop(0, nk, body, 0)
        plsc.subcore_barrier()
        pltpu.sync_copy(acc_sh.at[pl.ds(r0, rpt)],
                        part_hbm.at[pl.ds(c * NPAD + r0, rpt)])
        pltpu.sync_copy(hist, cnt_hbm.at[wid])

    return k(x, src2, dst2)


def _tc_dense(part, cnt, x_p, batch3, wlt, bl, wrt, w1t, b1, w2t, b2):
    d = x_p.shape[1]
    nblk = batch3.shape[0]
    blk = batch3.shape[1]

    def body(part_ref, cnt_ref, x_ref, b_ref, wl_ref, bl_ref, wr_ref,
             w1_ref, b1_ref, w2_ref, b2_ref, out_ref):
        i = pl.program_id(0)
        p = part_ref[0] + part_ref[1]
        cc = jnp.sum(cnt_ref[...], axis=0)
        ccb = lax.broadcast_in_dim(cc, (blk, d), (0,))
        mean = p / jnp.maximum(ccb, 1.0)
        h = (jnp.dot(mean, wl_ref[...], preferred_element_type=jnp.float32)
             + bl_ref[...]
             + jnp.dot(x_ref[...], wr_ref[...],
                       preferred_element_type=jnp.float32))
        h = jnp.maximum(h, 0.0)
        h = jnp.dot(h, w1_ref[...], preferred_element_type=jnp.float32) + b1_ref[...]
        h = jnp.maximum(h, 0.0)
        h = jnp.dot(h, w2_ref[...], preferred_element_type=jnp.float32) + b2_ref[...]
        h = 1.0 / (1.0 + jnp.exp(-h))
        bb = b_ref[0]

        @pl.when(i == 0)
        def _():
            out_ref[...] = jnp.full((NUM_G, d), -jnp.inf, jnp.float32)

        def gbody(g, carry):
            m = jnp.max(jnp.where(bb == g, h, -jnp.inf),
                        axis=0, keepdims=True)
            out_ref[pl.ds(g, 1), :] = jnp.maximum(out_ref[pl.ds(g, 1), :], m)
            return carry

        lax.fori_loop(0, NUM_G, gbody, 0)

    return pl.pallas_call(
        body,
        grid=(nblk,),
        in_specs=[
            pl.BlockSpec((2, blk, d), lambda i: (0, i, 0)),
            pl.BlockSpec((NW, blk), lambda i: (0, i)),
            pl.BlockSpec((blk, d), lambda i: (i, 0)),
            pl.BlockSpec((1, blk, 1), lambda i: (i, 0, 0)),
            pl.BlockSpec((d, d), lambda i: (0, 0)),
            pl.BlockSpec((1, d), lambda i: (0, 0)),
            pl.BlockSpec((d, d), lambda i: (0, 0)),
            pl.BlockSpec((d, 512), lambda i: (0, 0)),
            pl.BlockSpec((1, 512), lambda i: (0, 0)),
            pl.BlockSpec((512, d), lambda i: (0, 0)),
            pl.BlockSpec((1, d), lambda i: (0, 0)),
        ],
        out_specs=pl.BlockSpec((NUM_G, d), lambda i: (0, 0)),
        out_shape=jax.ShapeDtypeStruct((NUM_G, d), jnp.float32),
    )(part, cnt, x_p, batch3, wlt, bl, wrt, w1t, b1, w2t, b2)


@jax.jit
def _run(x, edge_index, batch, W_l, b_l, W_r, W1, b1, W2, b2):
    n, d = x.shape
    e = edge_index.shape[1]
    src2 = edge_index[0].reshape(e // CHUNK, CHUNK)
    dst2 = edge_index[1].reshape(e // CHUNK, CHUNK)
    part2, cnt3 = _sc_agg(x, src2, dst2)
    part = part2.reshape(2, NPAD, d)
    cnt = cnt3.reshape(NW, NPAD)
    x_p = jnp.zeros((NPAD, d), x.dtype).at[:n].set(x)
    batch_p = jnp.concatenate(
        [batch, jnp.full((NPAD - n,), NUM_G, jnp.int32)])
    blk = 1024
    batch3 = batch_p.reshape(NPAD // blk, blk, 1)
    return _tc_dense(part, cnt, x_p, batch3,
                     W_l.T, b_l.reshape(1, -1), W_r.T,
                     W1.T, b1.reshape(1, -1), W2.T, b2.reshape(1, -1))


def kernel(x, edge_index, batch, W_l, b_l, W_r, W1, b1, W2, b2):
    return _run(x, edge_index, batch, W_l, b_l, W_r, W1, b1, W2, b2)


# speedup vs baseline: 6.5849x; 1.2077x over previous
import functools

import jax
import jax.numpy as jnp
from jax import lax
from jax.experimental import pallas as pl
from jax.experimental.pallas import tpu as pltpu
from jax.experimental.pallas import tpu_sc as plsc

NUM_G = 64
CHUNK = 128
NPAD = 10240
NW = 32


def _sc_agg(x, src2, dst2):
    n, d = x.shape
    nch = src2.shape[0]
    rpt = NPAD // 16

    mesh = plsc.VectorSubcoreMesh(core_axis_name="c", subcore_axis_name="s")

    @functools.partial(
        pl.kernel,
        mesh=mesh,
        out_type=(
            jax.ShapeDtypeStruct((2 * NPAD, d), jnp.float32),
            jax.ShapeDtypeStruct((NW, NPAD // 128, 128), jnp.float32),
        ),
        scratch_types=[
            pltpu.VMEM_SHARED((NPAD, d), jnp.float32),
            pltpu.VMEM((NPAD // 128, 128), jnp.float32),
            pltpu.VMEM((CHUNK,), jnp.int32),
            pltpu.VMEM((CHUNK,), jnp.int32),
            pltpu.VMEM((CHUNK, d), jnp.float32),
            pltpu.VMEM((CHUNK,), jnp.int32),
            pltpu.VMEM((CHUNK,), jnp.int32),
            pltpu.VMEM((CHUNK, d), jnp.float32),
            pltpu.SemaphoreType.DMA,
            pltpu.SemaphoreType.DMA,
        ],
        compiler_params=pltpu.CompilerParams(
            use_tc_tiling_on_sc=False, needs_layout_passes=False),
    )
    def k(x_hbm, src_hbm, dst_hbm, part_hbm, cnt_hbm,
          acc_sh, hist, sidx, didx, rows, sidx1, didx1, rows1, sem, sem1):
        c = lax.axis_index("c")
        s = lax.axis_index("s")
        wid = s * 2 + c

        def fillz(i, carry):
            for j in range(d // 16):
                rows[i, pl.ds(j * 16, 16)] = jnp.zeros((16,), jnp.float32)
            return carry

        lax.fori_loop(0, CHUNK, fillz, 0)

        def fillh(i, carry):
            for j in range(128 // 16):
                hist[i, pl.ds(j * 16, 16)] = jnp.zeros((16,), jnp.float32)
            return carry

        lax.fori_loop(0, NPAD // 128, fillh, 0)

        r0 = s * rpt
        for j in range(rpt // CHUNK):
            pltpu.sync_copy(rows, acc_sh.at[pl.ds(r0 + j * CHUNK, CHUNK)])
        plsc.subcore_barrier()

        ones16 = jnp.ones((16,), jnp.float32)
        nk = nch // 32 + jnp.where(wid < nch % 32, 1, 0)
        nk2 = nk // 2

        def hist_update(dref):
            for j in range(CHUNK // 16):
                dj = dref[pl.ds(j * 16, 16)]
                plsc.addupdate_scatter(hist, [dj >> 7, dj & 127], ones16)

        def body(kk, carry):
            cid0 = wid + 32 * (2 * kk)
            cid1 = cid0 + 32
            pltpu.sync_copy(src_hbm.at[cid0], sidx)
            pltpu.sync_copy(dst_hbm.at[cid0], didx)
            g0 = pltpu.async_copy(x_hbm.at[sidx], rows, sem)
            pltpu.sync_copy(src_hbm.at[cid1], sidx1)
            pltpu.sync_copy(dst_hbm.at[cid1], didx1)
            g1 = pltpu.async_copy(x_hbm.at[sidx1], rows1, sem1)
            hist_update(didx)
            g0.wait()
            pltpu.sync_copy(rows, acc_sh.at[didx], add=True)
            hist_update(didx1)
            g1.wait()
            pltpu.sync_copy(rows1, acc_sh.at[didx1], add=True)
            return carry

        lax.fori_loop(0, nk2, body, 0)

        @pl.when(nk2 * 2 < nk)
        def _():
            cid = wid + 32 * (nk2 * 2)
            pltpu.sync_copy(src_hbm.at[cid], sidx)
            pltpu.sync_copy(dst_hbm.at[cid], didx)
            g = pltpu.async_copy(x_hbm.at[sidx], rows, sem)
            hist_update(didx)
            g.wait()
            pltpu.sync_copy(rows, acc_sh.at[didx], add=True)
        plsc.subcore_barrier()
        pltpu.sync_copy(acc_sh.at[pl.ds(r0, rpt)],
                        part_hbm.at[pl.ds(c * NPAD + r0, rpt)])
        pltpu.sync_copy(hist, cnt_hbm.at[wid])

    return k(x, src2, dst2)


def _tc_dense(part, cnt, x_p, batch3, wlt, bl, wrt, w1t, b1, w2t, b2):
    d = x_p.shape[1]
    nblk = batch3.shape[0]
    blk = batch3.shape[1]

    def body(part_ref, cnt_ref, x_ref, b_ref, wl_ref, bl_ref, wr_ref,
             w1_ref, b1_ref, w2_ref, b2_ref, out_ref):
        i = pl.program_id(0)
        p = part_ref[0] + part_ref[1]
        cc = jnp.sum(cnt_ref[...], axis=0)
        ccb = lax.broadcast_in_dim(cc, (blk, d), (0,))
        mean = p / jnp.maximum(ccb, 1.0)
        h = (jnp.dot(mean, wl_ref[...], preferred_element_type=jnp.float32)
             + bl_ref[...]
             + jnp.dot(x_ref[...], wr_ref[...],
                       preferred_element_type=jnp.float32))
        h = jnp.maximum(h, 0.0)
        h = jnp.dot(h, w1_ref[...], preferred_element_type=jnp.float32) + b1_ref[...]
        h = jnp.maximum(h, 0.0)
        h = jnp.dot(h, w2_ref[...], preferred_element_type=jnp.float32) + b2_ref[...]
        h = 1.0 / (1.0 + jnp.exp(-h))
        bb = b_ref[0]

        @pl.when(i == 0)
        def _():
            out_ref[...] = jnp.full((NUM_G, d), -jnp.inf, jnp.float32)

        def gbody(g, carry):
            m = jnp.max(jnp.where(bb == g, h, -jnp.inf),
                        axis=0, keepdims=True)
            out_ref[pl.ds(g, 1), :] = jnp.maximum(out_ref[pl.ds(g, 1), :], m)
            return carry

        lax.fori_loop(0, NUM_G, gbody, 0)

    return pl.pallas_call(
        body,
        grid=(nblk,),
        in_specs=[
            pl.BlockSpec((2, blk, d), lambda i: (0, i, 0)),
            pl.BlockSpec((NW, blk), lambda i: (0, i)),
            pl.BlockSpec((blk, d), lambda i: (i, 0)),
            pl.BlockSpec((1, blk, 1), lambda i: (i, 0, 0)),
            pl.BlockSpec((d, d), lambda i: (0, 0)),
            pl.BlockSpec((1, d), lambda i: (0, 0)),
            pl.BlockSpec((d, d), lambda i: (0, 0)),
            pl.BlockSpec((d, 512), lambda i: (0, 0)),
            pl.BlockSpec((1, 512), lambda i: (0, 0)),
            pl.BlockSpec((512, d), lambda i: (0, 0)),
            pl.BlockSpec((1, d), lambda i: (0, 0)),
        ],
        out_specs=pl.BlockSpec((NUM_G, d), lambda i: (0, 0)),
        out_shape=jax.ShapeDtypeStruct((NUM_G, d), jnp.float32),
    )(part, cnt, x_p, batch3, wlt, bl, wrt, w1t, b1, w2t, b2)


@jax.jit
def _run(x, edge_index, batch, W_l, b_l, W_r, W1, b1, W2, b2):
    n, d = x.shape
    e = edge_index.shape[1]
    src2 = edge_index[0].reshape(e // CHUNK, CHUNK)
    dst2 = edge_index[1].reshape(e // CHUNK, CHUNK)
    part2, cnt3 = _sc_agg(x, src2, dst2)
    part = part2.reshape(2, NPAD, d)
    cnt = cnt3.reshape(NW, NPAD)
    x_p = jnp.zeros((NPAD, d), x.dtype).at[:n].set(x)
    batch_p = jnp.concatenate(
        [batch, jnp.full((NPAD - n,), NUM_G, jnp.int32)])
    blk = 1024
    batch3 = batch_p.reshape(NPAD // blk, blk, 1)
    return _tc_dense(part, cnt, x_p, batch3,
                     W_l.T, b_l.reshape(1, -1), W_r.T,
                     W1.T, b1.reshape(1, -1), W2.T, b2.reshape(1, -1))


def kernel(x, edge_index, batch, W_l, b_l, W_r, W1, b1, W2, b2):
    return _run(x, edge_index, batch, W_l, b_l, W_r, W1, b1, W2, b2)


# speedup vs baseline: 6.6833x; 1.0149x over previous
import functools

import jax
import jax.numpy as jnp
from jax import lax
from jax.experimental import pallas as pl
from jax.experimental.pallas import tpu as pltpu
from jax.experimental.pallas import tpu_sc as plsc

NUM_G = 64
CHUNK = 128
NPAD = 10240
NW = 32


def _sc_agg(x, src2, dst2):
    n, d = x.shape
    nch = src2.shape[0]
    rpt = NPAD // 16

    mesh = plsc.VectorSubcoreMesh(core_axis_name="c", subcore_axis_name="s")

    @functools.partial(
        pl.kernel,
        mesh=mesh,
        out_type=(
            jax.ShapeDtypeStruct((2 * NPAD, d), jnp.float32),
            jax.ShapeDtypeStruct((NW, NPAD // 128, 128), jnp.float32),
        ),
        scratch_types=[
            pltpu.VMEM_SHARED((NPAD, d), jnp.float32),
            pltpu.VMEM((NPAD // 128, 128), jnp.float32),
            pltpu.VMEM((CHUNK,), jnp.int32),
            pltpu.VMEM((CHUNK,), jnp.int32),
            pltpu.VMEM((CHUNK, d), jnp.float32),
            pltpu.VMEM((CHUNK,), jnp.int32),
            pltpu.VMEM((CHUNK,), jnp.int32),
            pltpu.VMEM((CHUNK, d), jnp.float32),
            pltpu.SemaphoreType.DMA,
            pltpu.SemaphoreType.DMA,
            pltpu.SemaphoreType.DMA,
            pltpu.SemaphoreType.DMA,
        ],
        compiler_params=pltpu.CompilerParams(
            use_tc_tiling_on_sc=False, needs_layout_passes=False),
    )
    def k(x_hbm, src_hbm, dst_hbm, part_hbm, cnt_hbm,
          acc_sh, hist, sidx, didx, rows, sidx1, didx1, rows1,
          sem, sem1, semi, semi1):
        c = lax.axis_index("c")
        s = lax.axis_index("s")
        wid = s * 2 + c

        def fillz(i, carry):
            for j in range(d // 16):
                rows[i, pl.ds(j * 16, 16)] = jnp.zeros((16,), jnp.float32)
            return carry

        lax.fori_loop(0, CHUNK, fillz, 0)

        def fillh(i, carry):
            for j in range(128 // 16):
                hist[i, pl.ds(j * 16, 16)] = jnp.zeros((16,), jnp.float32)
            return carry

        lax.fori_loop(0, NPAD // 128, fillh, 0)

        r0 = s * rpt
        for j in range(rpt // CHUNK):
            pltpu.sync_copy(rows, acc_sh.at[pl.ds(r0 + j * CHUNK, CHUNK)])
        plsc.subcore_barrier()

        ones16 = jnp.ones((16,), jnp.float32)
        nk = nch // 32 + jnp.where(wid < nch % 32, 1, 0)
        nk2 = nk // 2

        def hist_update(dref):
            for j in range(CHUNK // 16):
                dj = dref[pl.ds(j * 16, 16)]
                plsc.addupdate_scatter(hist, [dj >> 7, dj & 127], ones16)

        def body(kk, carry):
            cid0 = wid + 32 * (2 * kk)
            cid1 = cid0 + 32
            ia0 = pltpu.async_copy(src_hbm.at[cid0], sidx, semi)
            ia1 = pltpu.async_copy(dst_hbm.at[cid0], didx, semi)
            ib0 = pltpu.async_copy(src_hbm.at[cid1], sidx1, semi1)
            ib1 = pltpu.async_copy(dst_hbm.at[cid1], didx1, semi1)
            ia0.wait()
            ia1.wait()
            g0 = pltpu.async_copy(x_hbm.at[sidx], rows, sem)
            ib0.wait()
            ib1.wait()
            g1 = pltpu.async_copy(x_hbm.at[sidx1], rows1, sem1)
            hist_update(didx)
            g0.wait()
            pltpu.sync_copy(rows, acc_sh.at[didx], add=True)
            hist_update(didx1)
            g1.wait()
            pltpu.sync_copy(rows1, acc_sh.at[didx1], add=True)
            return carry

        lax.fori_loop(0, nk2, body, 0)

        @pl.when(nk2 * 2 < nk)
        def _():
            cid = wid + 32 * (nk2 * 2)
            pltpu.sync_copy(src_hbm.at[cid], sidx)
            pltpu.sync_copy(dst_hbm.at[cid], didx)
            g = pltpu.async_copy(x_hbm.at[sidx], rows, sem)
            hist_update(didx)
            g.wait()
            pltpu.sync_copy(rows, acc_sh.at[didx], add=True)
        plsc.subcore_barrier()
        pltpu.sync_copy(acc_sh.at[pl.ds(r0, rpt)],
                        part_hbm.at[pl.ds(c * NPAD + r0, rpt)])
        pltpu.sync_copy(hist, cnt_hbm.at[wid])

    return k(x, src2, dst2)


def _tc_dense(part, cnt, x_p, batch3, wlt, bl, wrt, w1t, b1, w2t, b2):
    d = x_p.shape[1]
    nblk = batch3.shape[0]
    blk = batch3.shape[1]

    def body(part_ref, cnt_ref, x_ref, b_ref, wl_ref, bl_ref, wr_ref,
             w1_ref, b1_ref, w2_ref, b2_ref, out_ref):
        i = pl.program_id(0)
        p = part_ref[0] + part_ref[1]
        cc = jnp.sum(cnt_ref[...], axis=0)
        ccb = lax.broadcast_in_dim(cc, (blk, d), (0,))
        mean = p / jnp.maximum(ccb, 1.0)
        h = (jnp.dot(mean, wl_ref[...], preferred_element_type=jnp.float32)
             + bl_ref[...]
             + jnp.dot(x_ref[...], wr_ref[...],
                       preferred_element_type=jnp.float32))
        h = jnp.maximum(h, 0.0)
        h = jnp.dot(h, w1_ref[...], preferred_element_type=jnp.float32) + b1_ref[...]
        h = jnp.maximum(h, 0.0)
        h = jnp.dot(h, w2_ref[...], preferred_element_type=jnp.float32) + b2_ref[...]
        h = 1.0 / (1.0 + jnp.exp(-h))
        bb = b_ref[0]

        @pl.when(i == 0)
        def _():
            out_ref[...] = jnp.full((NUM_G, d), -jnp.inf, jnp.float32)

        def gbody(g, carry):
            m = jnp.max(jnp.where(bb == g, h, -jnp.inf),
                        axis=0, keepdims=True)
            out_ref[pl.ds(g, 1), :] = jnp.maximum(out_ref[pl.ds(g, 1), :], m)
            return carry

        lax.fori_loop(0, NUM_G, gbody, 0)

    return pl.pallas_call(
        body,
        grid=(nblk,),
        in_specs=[
            pl.BlockSpec((2, blk, d), lambda i: (0, i, 0)),
            pl.BlockSpec((NW, blk), lambda i: (0, i)),
            pl.BlockSpec((blk, d), lambda i: (i, 0)),
            pl.BlockSpec((1, blk, 1), lambda i: (i, 0, 0)),
            pl.BlockSpec((d, d), lambda i: (0, 0)),
            pl.BlockSpec((1, d), lambda i: (0, 0)),
            pl.BlockSpec((d, d), lambda i: (0, 0)),
            pl.BlockSpec((d, 512), lambda i: (0, 0)),
            pl.BlockSpec((1, 512), lambda i: (0, 0)),
            pl.BlockSpec((512, d), lambda i: (0, 0)),
            pl.BlockSpec((1, d), lambda i: (0, 0)),
        ],
        out_specs=pl.BlockSpec((NUM_G, d), lambda i: (0, 0)),
        out_shape=jax.ShapeDtypeStruct((NUM_G, d), jnp.float32),
    )(part, cnt, x_p, batch3, wlt, bl, wrt, w1t, b1, w2t, b2)


@jax.jit
def _run(x, edge_index, batch, W_l, b_l, W_r, W1, b1, W2, b2):
    n, d = x.shape
    e = edge_index.shape[1]
    src2 = edge_index[0].reshape(e // CHUNK, CHUNK)
    dst2 = edge_index[1].reshape(e // CHUNK, CHUNK)
    part2, cnt3 = _sc_agg(x, src2, dst2)
    part = part2.reshape(2, NPAD, d)
    cnt = cnt3.reshape(NW, NPAD)
    x_p = jnp.zeros((NPAD, d), x.dtype).at[:n].set(x)
    batch_p = jnp.concatenate(
        [batch, jnp.full((NPAD - n,), NUM_G, jnp.int32)])
    blk = 1024
    batch3 = batch_p.reshape(NPAD // blk, blk, 1)
    return _tc_dense(part, cnt, x_p, batch3,
                     W_l.T, b_l.reshape(1, -1), W_r.T,
                     W1.T, b1.reshape(1, -1), W2.T, b2.reshape(1, -1))


def kernel(x, edge_index, batch, W_l, b_l, W_r, W1, b1, W2, b2):
    return _run(x, edge_index, batch, W_l, b_l, W_r, W1, b1, W2, b2)
